# Initial kernel scaffold; baseline (speedup 1.0000x reference)
import jax
import jax.numpy as jnp
from jax.experimental import pallas as pl


def kernel(node_features, edge_features, W1, b1, W2, b2, W3, b3, gamma, beta, receivers, num_nodes):
    raise NotImplementedError("write your pallas kernel here")


# speedup vs baseline: 1.7573x; 1.7573x over previous
import functools

import jax
import jax.numpy as jnp
from jax import lax
from jax.experimental import pallas as pl
from jax.experimental.pallas import tpu as pltpu
from jax.experimental.pallas import tpu_sc as plsc

_NC = 2
_NS = 16
_LANES = 16

_SPLIT = 5200
_TROWS = 5248
_TRASH = 5200

_CHUNK = 80


def _scatter_add_call(edge_features, receivers):
    e, ed = edge_features.shape
    assert ed == _LANES
    assert e % (_NS * _CHUNK) == 0, (e, _NS * _CHUNK)
    edges_per_tile = e // _NS
    nchunk = edges_per_tile // _CHUNK
    srows = _TROWS // _NS
    nsub = _CHUNK // _LANES

    mesh = plsc.VectorSubcoreMesh(core_axis_name="c", subcore_axis_name="s")

    @functools.partial(
        pl.kernel,
        out_type=[
            jax.ShapeDtypeStruct((_NC, _TROWS, ed), jnp.float32),
            jax.ShapeDtypeStruct((_NC, _TROWS, ed), jnp.float32),
        ],
        mesh=mesh,
        scratch_types=[
            pltpu.VMEM((_CHUNK, ed), jnp.float32),
            pltpu.VMEM((_CHUNK,), jnp.int32),
            pltpu.VMEM((_CHUNK,), jnp.int32),
            pltpu.VMEM((_CHUNK, ed), jnp.float32),
            pltpu.VMEM((srows, ed), jnp.float32),
            pltpu.VMEM_SHARED((_TROWS, ed), jnp.float32),
            pltpu.VMEM_SHARED((_TROWS, ed), jnp.float32),
        ],
    )
    def scatter_kernel(edges_hbm, recv_hbm, feat_out, deg_out,
                       edge_v, idx_v, idx2_v, ones_v, stage_v, sfeat, sdeg):
        cid = lax.axis_index("c")
        sid = lax.axis_index("s")
        base_node = cid * _SPLIT

        def zero_stage(i, carry):
            stage_v[i, :] = jnp.zeros((ed,), jnp.float32)
            return carry

        lax.fori_loop(0, srows, zero_stage, 0)

        def fill_ones(i, carry):
            ones_v[i, :] = jnp.ones((ed,), jnp.float32)
            return carry

        lax.fori_loop(0, _CHUNK, fill_ones, 0)

        r0 = sid * srows
        pltpu.sync_copy(stage_v, sfeat.at[pl.ds(r0, srows)])
        pltpu.sync_copy(stage_v, sdeg.at[pl.ds(r0, srows)])
        plsc.subcore_barrier()

        ebase = sid * edges_per_tile

        def chunk_body(j, carry):
            off = ebase + j * _CHUNK
            pltpu.sync_copy(recv_hbm.at[pl.ds(off, _CHUNK)], idx_v)
            pltpu.sync_copy(edges_hbm.at[pl.ds(off, _CHUNK)], edge_v)

            for k in range(nsub):
                iv = idx_v[pl.ds(k * _LANES, _LANES)]
                t = iv - base_node
                oob = jnp.logical_or(t < 0, t >= _SPLIT)
                idx2_v[pl.ds(k * _LANES, _LANES)] = jnp.where(
                    oob, jnp.full((_LANES,), _TRASH, jnp.int32), t)

            pltpu.sync_copy(edge_v, sfeat.at[idx2_v], add=True)
            pltpu.sync_copy(ones_v, sdeg.at[idx2_v], add=True)
            return carry

        lax.fori_loop(0, nchunk, chunk_body, 0)
        plsc.subcore_barrier()

        pltpu.sync_copy(sfeat.at[pl.ds(r0, srows)], stage_v)
        pltpu.sync_copy(stage_v, feat_out.at[cid, pl.ds(r0, srows)])
        pltpu.sync_copy(sdeg.at[pl.ds(r0, srows)], stage_v)
        pltpu.sync_copy(stage_v, deg_out.at[cid, pl.ds(r0, srows)])

    return scatter_kernel(edge_features, receivers)


def _mlp_call(node_features, feat_p, deg_p, bias_row,
              w1a, w1b, w2, w3, b1, b2, b3, gamma, beta):
    n, nd = node_features.shape
    ed = feat_p.shape[2]
    hid = w2.shape[0]
    blk = 400
    assert n % blk == 0 and _SPLIT % blk == 0
    grid = (n // blk,)
    sblk = _SPLIT // blk

    def body(nf_ref, fp_ref, dp_ref, bias_ref, w1a_ref, w1b_ref, w2_ref,
             w3_ref, b1_ref, b2_ref, b3_ref, g_ref, bt_ref, out_ref):
        f = fp_ref[0]
        d = dp_ref[0][:, 0:1]
        agg = (f + bias_ref[...]) / jnp.maximum(d, 1.0)
        h = jnp.dot(nf_ref[...], w1a_ref[...],
                    preferred_element_type=jnp.float32)
        h = h + jnp.dot(agg, w1b_ref[...],
                        preferred_element_type=jnp.float32)
        h = jnp.maximum(h + b1_ref[...], 0.0)
        h = jnp.dot(h, w2_ref[...], preferred_element_type=jnp.float32)
        h = jnp.maximum(h + b2_ref[...], 0.0)
        h = jnp.dot(h, w3_ref[...], preferred_element_type=jnp.float32)
        h = h + b3_ref[...]
        mu = jnp.mean(h, axis=-1, keepdims=True)
        var = jnp.mean((h - mu) ** 2, axis=-1, keepdims=True)
        out_ref[...] = (h - mu) * lax.rsqrt(var + 1e-5) * g_ref[...] + bt_ref[...]

    full = lambda i: (0, 0)

    def win_map(i):
        lo = i < sblk
        return (jnp.where(lo, 0, 1), jnp.where(lo, i, i - sblk), 0)

    return pl.pallas_call(
        body,
        grid=grid,
        in_specs=[
            pl.BlockSpec((blk, nd), lambda i: (i, 0)),
            pl.BlockSpec((1, blk, ed), win_map),
            pl.BlockSpec((1, blk, ed), win_map),
            pl.BlockSpec((1, ed), full),
            pl.BlockSpec((nd, hid), full),
            pl.BlockSpec((ed, hid), full),
            pl.BlockSpec((hid, hid), full),
            pl.BlockSpec((hid, hid), full),
            pl.BlockSpec((1, hid), full),
            pl.BlockSpec((1, hid), full),
            pl.BlockSpec((1, hid), full),
            pl.BlockSpec((1, hid), full),
            pl.BlockSpec((1, hid), full),
        ],
        out_specs=pl.BlockSpec((blk, hid), lambda i: (i, 0)),
        out_shape=jax.ShapeDtypeStruct((n, hid), jnp.float32),
    )(node_features, feat_p, deg_p, bias_row, w1a, w1b, w2, w3,
      b1, b2, b3, gamma, beta)


def kernel(node_features, edge_features, W1, b1, W2, b2, W3, b3,
           gamma, beta, receivers, num_nodes):
    n, nd = node_features.shape
    ed = edge_features.shape[1]

    feat_p, deg_p = _scatter_add_call(edge_features, receivers)

    bias_row = (jnp.asarray(num_nodes, jnp.float32) - n) * jnp.ones(
        (1, ed), jnp.float32)
    hid = W2.shape[0]
    return _mlp_call(
        node_features, feat_p, deg_p, bias_row,
        W1[:nd], W1[nd:], W2, W3,
        b1.reshape(1, hid), b2.reshape(1, hid), b3.reshape(1, hid),
        gamma.reshape(1, hid), beta.reshape(1, hid),
    )


# speedup vs baseline: 1.9732x; 1.1229x over previous
import functools

import jax
import jax.numpy as jnp
from jax import lax
from jax.experimental import pallas as pl
from jax.experimental.pallas import tpu as pltpu
from jax.experimental.pallas import tpu_sc as plsc

_NC = 2
_NS = 16
_LANES = 16

_SPLIT = 5200
_TROWS = 5248
_TRASH = 5200

_CHUNK = 80
_GRP = 2


def _scatter_add_call(edge_features, receivers):
    e, ed = edge_features.shape
    assert ed == _LANES
    sbc = 8
    sbe = sbc * _CHUNK
    nsb = e // sbe
    sb_pt = nsb // _NS
    sb_extra = nsb % _NS
    tail_e = e - nsb * sbe
    tail_c = tail_e // _CHUNK
    assert tail_c * _CHUNK == tail_e and tail_c <= sbc
    assert sbc % _GRP == 0
    srows = _TROWS // _NS
    nsl = 1
    assert srows % nsl == 0
    sub = srows // nsl
    nsub = _CHUNK // _LANES

    mesh = plsc.VectorSubcoreMesh(core_axis_name="c", subcore_axis_name="s")

    @functools.partial(
        pl.kernel,
        out_type=[
            jax.ShapeDtypeStruct((_NC, _TROWS, ed), jnp.float32),
            jax.ShapeDtypeStruct((_NC, _TROWS, ed), jnp.float32),
        ],
        mesh=mesh,
        scratch_types=[
            pltpu.VMEM((_GRP * _CHUNK, ed), jnp.float32),
            pltpu.VMEM((sbe,), jnp.int32),
        ] + [
            pltpu.VMEM((_CHUNK,), jnp.int32) for _ in range(sbc)
        ] + [
            pltpu.VMEM((_CHUNK, ed), jnp.float32),
            pltpu.VMEM((sub, ed), jnp.float32),
            pltpu.VMEM_SHARED((_TROWS, ed), jnp.float32),
            pltpu.VMEM_SHARED((_TROWS, ed), jnp.float32),
        ],
    )
    def scatter_kernel(edges_hbm, recv_hbm, feat_out, deg_out,
                       edge_v, raw_v, *rest):
        rbufs = rest[:sbc]
        ones_v, stage_v, sfeat, sdeg = rest[sbc:]
        cid = lax.axis_index("c")
        sid = lax.axis_index("s")
        base_node = cid * _SPLIT
        sb_start = sb_pt * sid + jnp.minimum(sid, sb_extra)
        nsb_t = sb_pt + jnp.where(sid < sb_extra, 1, 0)

        def zero_stage(i, carry):
            stage_v[i, :] = jnp.zeros((ed,), jnp.float32)
            return carry

        lax.fori_loop(0, sub, zero_stage, 0)

        def fill_ones(i, carry):
            ones_v[i, :] = jnp.ones((ed,), jnp.float32)
            return carry

        lax.fori_loop(0, _CHUNK, fill_ones, 0)

        r0 = sid * srows
        for z in range(nsl):
            pltpu.sync_copy(stage_v, sfeat.at[pl.ds(r0 + z * sub, sub)])
            pltpu.sync_copy(stage_v, sdeg.at[pl.ds(r0 + z * sub, sub)])
        plsc.subcore_barrier()

        trash16 = jnp.full((_LANES,), _TRASH, jnp.int32)

        def remap(iv):
            t = iv - base_node
            oob = jnp.logical_or(t < 0, t >= _SPLIT)
            return jnp.where(oob, trash16, t)

        def do_chunks(eoff, nchunks_here):
            for q in range(nchunks_here):
                for k in range(nsub):
                    iv = raw_v[pl.ds(q * _CHUNK + k * _LANES, _LANES)]
                    rbufs[q][pl.ds(k * _LANES, _LANES)] = remap(iv)
            for half in range((nchunks_here + _GRP - 1) // _GRP):
                pltpu.sync_copy(
                    edges_hbm.at[pl.ds(eoff + half * _GRP * _CHUNK,
                                       _GRP * _CHUNK)], edge_v)
                for q2 in range(min(_GRP, nchunks_here - half * _GRP)):
                    qq = half * _GRP + q2
                    pltpu.sync_copy(edge_v.at[pl.ds(q2 * _CHUNK, _CHUNK)],
                                    sfeat.at[rbufs[qq]], add=True)
                    pltpu.sync_copy(ones_v, sdeg.at[rbufs[qq]], add=True)

        def sb_body(s, carry):
            eoff = pl.multiple_of((sb_start + s) * sbe, sbe)
            pltpu.sync_copy(recv_hbm.at[pl.ds(eoff, sbe)], raw_v)
            do_chunks(eoff, sbc)
            return carry

        lax.fori_loop(0, nsb_t, sb_body, 0)

        if tail_c:
            @pl.when(sid == _NS - 1)
            def _():
                toff = nsb * sbe
                pltpu.sync_copy(recv_hbm.at[pl.ds(toff, tail_c * _CHUNK)],
                                raw_v.at[pl.ds(0, tail_c * _CHUNK)])
                do_chunks(toff, tail_c)

        plsc.subcore_barrier()

        for z in range(nsl):
            zoff = r0 + z * sub
            pltpu.sync_copy(sfeat.at[pl.ds(zoff, sub)], stage_v)
            pltpu.sync_copy(stage_v, feat_out.at[cid, pl.ds(zoff, sub)])
            pltpu.sync_copy(sdeg.at[pl.ds(zoff, sub)], stage_v)
            pltpu.sync_copy(stage_v, deg_out.at[cid, pl.ds(zoff, sub)])

    return scatter_kernel(edge_features, receivers)


def _mlp_call(node_features, feat_p, deg_p, bias_row,
              w1a, w1b, w2, w3, b1, b2, b3, gamma, beta):
    n, nd = node_features.shape
    ed = feat_p.shape[2]
    hid = w2.shape[0]
    blk = 400
    assert n % blk == 0 and _SPLIT % blk == 0
    grid = (n // blk,)
    sblk = _SPLIT // blk

    def body(nf_ref, fp_ref, dp_ref, bias_ref, w1a_ref, w1b_ref, w2_ref,
             w3_ref, b1_ref, b2_ref, b3_ref, g_ref, bt_ref, out_ref):
        f = fp_ref[0]
        d = dp_ref[0][:, 0:1]
        agg = (f + bias_ref[...]) / jnp.maximum(d, 1.0)
        h = jnp.dot(nf_ref[...], w1a_ref[...],
                    preferred_element_type=jnp.float32)
        h = h + jnp.dot(agg, w1b_ref[...],
                        preferred_element_type=jnp.float32)
        h = jnp.maximum(h + b1_ref[...], 0.0)
        h = jnp.dot(h, w2_ref[...], preferred_element_type=jnp.float32)
        h = jnp.maximum(h + b2_ref[...], 0.0)
        h = jnp.dot(h, w3_ref[...], preferred_element_type=jnp.float32)
        h = h + b3_ref[...]
        mu = jnp.mean(h, axis=-1, keepdims=True)
        var = jnp.mean((h - mu) ** 2, axis=-1, keepdims=True)
        out_ref[...] = (h - mu) * lax.rsqrt(var + 1e-5) * g_ref[...] + bt_ref[...]

    full = lambda i: (0, 0)

    def win_map(i):
        lo = i < sblk
        return (jnp.where(lo, 0, 1), jnp.where(lo, i, i - sblk), 0)

    return pl.pallas_call(
        body,
        grid=grid,
        in_specs=[
            pl.BlockSpec((blk, nd), lambda i: (i, 0)),
            pl.BlockSpec((1, blk, ed), win_map),
            pl.BlockSpec((1, blk, ed), win_map),
            pl.BlockSpec((1, ed), full),
            pl.BlockSpec((nd, hid), full),
            pl.BlockSpec((ed, hid), full),
            pl.BlockSpec((hid, hid), full),
            pl.BlockSpec((hid, hid), full),
            pl.BlockSpec((1, hid), full),
            pl.BlockSpec((1, hid), full),
            pl.BlockSpec((1, hid), full),
            pl.BlockSpec((1, hid), full),
            pl.BlockSpec((1, hid), full),
        ],
        out_specs=pl.BlockSpec((blk, hid), lambda i: (i, 0)),
        out_shape=jax.ShapeDtypeStruct((n, hid), jnp.float32),
    )(node_features, feat_p, deg_p, bias_row, w1a, w1b, w2, w3,
      b1, b2, b3, gamma, beta)


def kernel(node_features, edge_features, W1, b1, W2, b2, W3, b3,
           gamma, beta, receivers, num_nodes):
    n, nd = node_features.shape
    ed = edge_features.shape[1]

    feat_p, deg_p = _scatter_add_call(edge_features, receivers)

    bias_row = (jnp.asarray(num_nodes, jnp.float32) - n) * jnp.ones(
        (1, ed), jnp.float32)
    hid = W2.shape[0]
    return _mlp_call(
        node_features, feat_p, deg_p, bias_row,
        W1[:nd], W1[nd:], W2, W3,
        b1.reshape(1, hid), b2.reshape(1, hid), b3.reshape(1, hid),
        gamma.reshape(1, hid), beta.reshape(1, hid),
    )
